# Initial kernel scaffold; baseline (speedup 1.0000x reference)
import jax
import jax.numpy as jnp
from jax.experimental import pallas as pl


def kernel(point_cloud, category_ids, W1, b1, g1, bb1, W2, b2, g2, bb2, W3, b3, g3, bb3, W4, b4, g4, bb4, W5, b5, g5, bb5, W6, b6, g6, bb6, W7, b7, g7, bb7, cat_bias):
    raise NotImplementedError("write your pallas kernel here")


# speedup vs baseline: 7.8180x; 7.8180x over previous
import functools

import jax
import jax.numpy as jnp
from jax import lax
from jax.experimental import pallas as pl
from jax.experimental.pallas import tpu as pltpu
from jax.experimental.pallas import tpu_sc as plsc

_B, _N, _K = 8, 1024, 16
_BN = _B * _N
_NC, _NS = 2, 16
_NW = _NC * _NS
_PTS_W = _BN // _NW
_GCH = 8


def _dot(a, b):
    return jnp.dot(a.astype(jnp.bfloat16), b.astype(jnp.bfloat16),
                   preferred_element_type=jnp.float32)


def _bn(y, g, b, eps=1e-5):
    mu = jnp.mean(y, axis=0, keepdims=True)
    var = jnp.mean((y - mu) ** 2, axis=0, keepdims=True)
    return (y - mu) / jnp.sqrt(var + eps) * g + b


def _relu(x):
    return jnp.maximum(x, 0.0)


def _enc_body(x_ref, w1_ref, p1_ref, w2_ref, p2_ref, lf_ref):
    x = x_ref[...]
    h = _dot(x, w1_ref[...])
    h = _relu(_bn(h + p1_ref[0:1, :], p1_ref[1:2, :], p1_ref[2:3, :]))
    h2 = _dot(h, w2_ref[...])
    lf_ref[...] = _relu(_bn(h2 + p2_ref[0:1, :], p2_ref[1:2, :], p2_ref[2:3, :]))


def _knn_body(lf_ref, idx_ref):
    b = pl.program_id(0)
    x = lf_ref[0]
    sq = jnp.sum(x * x, axis=1)
    xb = x.astype(jnp.bfloat16)
    dot = lax.dot_general(xb, xb, (((1,), (1,)), ((), ())),
                          preferred_element_type=jnp.float32)
    dist = sq[:, None] + sq[None, :] - 2.0 * dot
    iota = lax.broadcasted_iota(jnp.int32, (_N, _N), 1)
    cols = []
    for _ in range(_K):
        m = jnp.min(dist, axis=1, keepdims=True)
        am = jnp.min(jnp.where(dist == m, iota, _N), axis=1, keepdims=True)
        cols.append(am)
        dist = jnp.where(iota == am, jnp.inf, dist)
    idx_ref[0] = jnp.concatenate(cols, axis=1) + b * _N


def _pool_body(lf_hbm, idx_hbm, ef_hbm, idx_v, rows_v, out_v, sem):
    wid = lax.axis_index("s") * _NC + lax.axis_index("c")
    base = wid * _PTS_W

    def chunk(ci, _):
        p0 = base + ci * _GCH
        pltpu.sync_copy(idx_hbm.at[pl.ds(p0 * _K, _GCH * _K)], idx_v)
        pltpu.async_copy(lf_hbm.at[idx_v], rows_v, sem).wait()
        for p in range(_GCH):
            for d in range(8):
                acc = rows_v[p * _K, pl.ds(d * 16, 16)]
                for r in range(1, _K):
                    acc = jnp.maximum(acc, rows_v[p * _K + r, pl.ds(d * 16, 16)])
                out_v[p, pl.ds(d * 16, 16)] = acc
        pltpu.sync_copy(out_v, ef_hbm.at[pl.ds(p0, _GCH)])
        return 0

    lax.fori_loop(0, _PTS_W // _GCH, chunk, 0)


def _tail_body(lf_ref, ef_ref, w3a_ref, w3b_ref, p3_ref, w4_ref, p4_ref,
               w5_ref, p5_ref, w6_ref, p6_ref, w7a_ref, w7b_ref, p7_ref,
               cid_ref, cbias_ref, out_ref):
    lf = lf_ref[...]
    ef = ef_ref[...]
    h = _dot(lf, w3a_ref[...]) + _dot(ef, w3b_ref[...])
    el = _relu(_bn(h + p3_ref[0:1, :], p3_ref[1:2, :], p3_ref[2:3, :]))
    el = _dot(el, w4_ref[...])
    el = _relu(_bn(el + p4_ref[0:1, :], p4_ref[1:2, :], p4_ref[2:3, :]))
    gin = jnp.concatenate(
        [jnp.max(lax.slice_in_dim(el, bb * _N, (bb + 1) * _N, axis=0),
                 axis=0, keepdims=True) for bb in range(_B)], axis=0)
    gf = _dot(gin, w5_ref[...])
    gf = _relu(_bn(gf + p5_ref[0:1, :], p5_ref[1:2, :], p5_ref[2:3, :]))
    gf = _dot(gf, w6_ref[...])
    gf = _relu(_bn(gf + p6_ref[0:1, :], p6_ref[1:2, :], p6_ref[2:3, :]))
    gb = _dot(gf, w7b_ref[...])
    feat = _dot(el, w7a_ref[...])
    feat = feat + jnp.concatenate(
        [jnp.broadcast_to(lax.slice_in_dim(gb, bb, bb + 1, axis=0), (_N, 512))
         for bb in range(_B)], axis=0)
    feat = _relu(_bn(feat + p7_ref[0:1, :], p7_ref[1:2, :], p7_ref[2:3, :]))
    onehot = (cid_ref[...] == lax.broadcasted_iota(jnp.int32, (_B, 10), 1))
    cb = jnp.dot(onehot.astype(jnp.float32), cbias_ref[...],
                 preferred_element_type=jnp.float32)
    out_ref[...] = feat + 0.1 * jnp.concatenate(
        [jnp.broadcast_to(lax.slice_in_dim(cb, bb, bb + 1, axis=0), (_N, 512))
         for bb in range(_B)], axis=0)


def _pack(b, g, bb):
    return jnp.stack([b, g, bb], axis=0)


def kernel(point_cloud, category_ids, W1, b1, g1, bb1, W2, b2, g2, bb2,
           W3, b3, g3, bb3, W4, b4, g4, bb4, W5, b5, g5, bb5,
           W6, b6, g6, bb6, W7, b7, g7, bb7, cat_bias):
    x = point_cloud.reshape(_BN, 3)

    lf = pl.pallas_call(
        _enc_body,
        out_shape=jax.ShapeDtypeStruct((_BN, 128), jnp.float32),
    )(x, W1.T, _pack(b1, g1, bb1), W2.T, _pack(b2, g2, bb2))

    idx = pl.pallas_call(
        _knn_body,
        grid=(_B,),
        in_specs=[pl.BlockSpec((1, _N, 128), lambda b: (b, 0, 0))],
        out_specs=pl.BlockSpec((1, _N, _K), lambda b: (b, 0, 0)),
        out_shape=jax.ShapeDtypeStruct((_B, _N, _K), jnp.int32),
    )(lf.reshape(_B, _N, 128))

    mesh = plsc.VectorSubcoreMesh(core_axis_name="c", subcore_axis_name="s")
    ef = pl.kernel(
        _pool_body,
        out_type=jax.ShapeDtypeStruct((_BN, 128), jnp.float32),
        mesh=mesh,
        scratch_types=[
            pltpu.VMEM((_GCH * _K,), jnp.int32),
            pltpu.VMEM((_GCH * _K, 128), jnp.float32),
            pltpu.VMEM((_GCH, 128), jnp.float32),
            pltpu.SemaphoreType.DMA,
        ],
    )(lf, idx.reshape(_BN * _K))

    out = pl.pallas_call(
        _tail_body,
        out_shape=jax.ShapeDtypeStruct((_BN, 512), jnp.float32),
    )(lf, ef, W3[:, :128].T, W3[:, 128:].T, _pack(b3, g3, bb3),
      W4.T, _pack(b4, g4, bb4), W5.T, _pack(b5, g5, bb5),
      W6.T, _pack(b6, g6, bb6), W7[:, :128].T, W7[:, 128:].T,
      _pack(b7, g7, bb7), category_ids.reshape(_B, 1), cat_bias)

    return out.reshape(_B, _N, 512)


# speedup vs baseline: 10.8196x; 1.3839x over previous
import functools

import jax
import jax.numpy as jnp
from jax import lax
from jax.experimental import pallas as pl
from jax.experimental.pallas import tpu as pltpu
from jax.experimental.pallas import tpu_sc as plsc

_B, _N, _K = 8, 1024, 16
_BN = _B * _N
_NC, _NS = 2, 16
_NW = _NC * _NS
_PTS_W = _BN // _NW
_GCH = 8


def _dot(a, b):
    return jnp.dot(a.astype(jnp.bfloat16), b.astype(jnp.bfloat16),
                   preferred_element_type=jnp.float32)


def _bn(y, g, b, eps=1e-5):
    mu = jnp.mean(y, axis=0, keepdims=True)
    var = jnp.mean((y - mu) ** 2, axis=0, keepdims=True)
    return (y - mu) / jnp.sqrt(var + eps) * g + b


def _relu(x):
    return jnp.maximum(x, 0.0)


def _enc_body(x_ref, w1_ref, p1_ref, w2_ref, p2_ref, lf_ref, lfb_ref):
    x = x_ref[...]
    h = _dot(x, w1_ref[...])
    h = _relu(_bn(h + p1_ref[0:1, :], p1_ref[1:2, :], p1_ref[2:3, :]))
    h2 = _dot(h, w2_ref[...])
    lf = _relu(_bn(h2 + p2_ref[0:1, :], p2_ref[1:2, :], p2_ref[2:3, :]))
    lf_ref[...] = lf
    lfb_ref[...] = lf.astype(jnp.bfloat16)


def _knn_body(lf_ref, idx_ref):
    b = pl.program_id(0)
    x = lf_ref[0]
    sq = jnp.sum(x * x, axis=1)
    xb = x.astype(jnp.bfloat16)
    dot = lax.dot_general(xb, xb, (((1,), (1,)), ((), ())),
                          preferred_element_type=jnp.float32)
    dist = sq[:, None] + sq[None, :] - 2.0 * dot
    key = lax.bitcast_convert_type(jnp.maximum(dist, 0.0), jnp.int32)
    key = jnp.bitwise_or(jnp.bitwise_and(key, ~jnp.int32(1023)),
                         lax.broadcasted_iota(jnp.int32, (_N, _N), 1))
    prev = jnp.full((_N, 1), -1, jnp.int32)
    big = jnp.int32(0x7FFFFFFF)
    cols = []
    for _ in range(_K):
        prev = jnp.min(jnp.where(key > prev, key, big), axis=1, keepdims=True)
        cols.append(jnp.bitwise_and(prev, 1023))
    idx_ref[0] = jnp.concatenate(cols, axis=1) + b * _N


_NCH = _PTS_W // _GCH


def _pool_chunk(rows_v, out_v):
    for p in range(_GCH):
        for d in range(8):
            acc = rows_v[p * _K, pl.ds(d * 16, 16)]
            for r in range(1, _K):
                acc = jnp.maximum(acc, rows_v[p * _K + r, pl.ds(d * 16, 16)])
            out_v[p, pl.ds(d * 16, 16)] = acc


def _pool_body(lf_hbm, idx_hbm, ef_hbm, idx_v, rows0_v, rows1_v, out_v,
               sem0, sem1):
    wid = lax.axis_index("s") * _NC + lax.axis_index("c")
    base = wid * _PTS_W
    rows = (rows0_v, rows1_v)
    sems = (sem0, sem1)

    pltpu.sync_copy(idx_hbm.at[wid], idx_v)
    pltpu.async_copy(lf_hbm.at[idx_v.at[0]], rows0_v, sem0)

    def step(i, _):
        for hb in range(2):
            ci = i * 2 + hb
            nxt = ci + 1

            @pl.when(nxt < _NCH)
            def _():
                pltpu.async_copy(lf_hbm.at[idx_v.at[nxt]], rows[1 - hb],
                                 sems[1 - hb])

            pltpu.make_async_copy(lf_hbm.at[idx_v.at[ci]], rows[hb],
                                  sems[hb]).wait()
            _pool_chunk(rows[hb], out_v)
            pltpu.sync_copy(out_v, ef_hbm.at[pl.ds(base + ci * _GCH, _GCH)])
        return 0

    lax.fori_loop(0, _NCH // 2, step, 0)


def _tail_body(lf_ref, ef_ref, w3a_ref, w3b_ref, p3_ref, w4_ref, p4_ref,
               w5_ref, p5_ref, w6_ref, p6_ref, w7a_ref, w7b_ref, p7_ref,
               cid_ref, cbias_ref, out_ref):
    lf = lf_ref[...]
    ef = ef_ref[...]
    h = _dot(lf, w3a_ref[...]) + _dot(ef, w3b_ref[...])
    el = _relu(_bn(h + p3_ref[0:1, :], p3_ref[1:2, :], p3_ref[2:3, :]))
    el = _dot(el, w4_ref[...])
    el = _relu(_bn(el + p4_ref[0:1, :], p4_ref[1:2, :], p4_ref[2:3, :]))
    gin = jnp.concatenate(
        [jnp.max(lax.slice_in_dim(el, bb * _N, (bb + 1) * _N, axis=0),
                 axis=0, keepdims=True) for bb in range(_B)], axis=0)
    gf = _dot(gin, w5_ref[...])
    gf = _relu(_bn(gf + p5_ref[0:1, :], p5_ref[1:2, :], p5_ref[2:3, :]))
    gf = _dot(gf, w6_ref[...])
    gf = _relu(_bn(gf + p6_ref[0:1, :], p6_ref[1:2, :], p6_ref[2:3, :]))
    gb = _dot(gf, w7b_ref[...])
    feat = _dot(el, w7a_ref[...])
    feat = feat + jnp.concatenate(
        [jnp.broadcast_to(lax.slice_in_dim(gb, bb, bb + 1, axis=0), (_N, 512))
         for bb in range(_B)], axis=0)
    feat = _relu(_bn(feat + p7_ref[0:1, :], p7_ref[1:2, :], p7_ref[2:3, :]))
    onehot = (cid_ref[...] == lax.broadcasted_iota(jnp.int32, (_B, 10), 1))
    cb = jnp.dot(onehot.astype(jnp.float32), cbias_ref[...],
                 preferred_element_type=jnp.float32)
    out_ref[...] = feat + 0.1 * jnp.concatenate(
        [jnp.broadcast_to(lax.slice_in_dim(cb, bb, bb + 1, axis=0), (_N, 512))
         for bb in range(_B)], axis=0)


def _pack(b, g, bb):
    return jnp.stack([b, g, bb], axis=0)


def kernel(point_cloud, category_ids, W1, b1, g1, bb1, W2, b2, g2, bb2,
           W3, b3, g3, bb3, W4, b4, g4, bb4, W5, b5, g5, bb5,
           W6, b6, g6, bb6, W7, b7, g7, bb7, cat_bias):
    x = point_cloud.reshape(_BN, 3)

    lf, lfb = pl.pallas_call(
        _enc_body,
        out_shape=[jax.ShapeDtypeStruct((_BN, 128), jnp.float32),
                   jax.ShapeDtypeStruct((_BN, 128), jnp.bfloat16)],
    )(x, W1.T, _pack(b1, g1, bb1), W2.T, _pack(b2, g2, bb2))

    idx = pl.pallas_call(
        _knn_body,
        grid=(_B,),
        in_specs=[pl.BlockSpec((1, _N, 128), lambda b: (b, 0, 0))],
        out_specs=pl.BlockSpec((1, _N, _K), lambda b: (b, 0, 0)),
        out_shape=jax.ShapeDtypeStruct((_B, _N, _K), jnp.int32),
    )(lf.reshape(_B, _N, 128))

    mesh = plsc.VectorSubcoreMesh(core_axis_name="c", subcore_axis_name="s")
    ef = pl.kernel(
        _pool_body,
        out_type=jax.ShapeDtypeStruct((_BN, 128), jnp.float32),
        mesh=mesh,
        scratch_types=[
            pltpu.VMEM((_NCH, _GCH * _K), jnp.int32),
            pltpu.VMEM((_GCH * _K, 128), jnp.float32),
            pltpu.VMEM((_GCH * _K, 128), jnp.float32),
            pltpu.VMEM((_GCH, 128), jnp.float32),
            pltpu.SemaphoreType.DMA,
            pltpu.SemaphoreType.DMA,
        ],
    )(lf, idx.reshape(_NW, _NCH, _GCH * _K))

    out = pl.pallas_call(
        _tail_body,
        out_shape=jax.ShapeDtypeStruct((_BN, 512), jnp.float32),
    )(lf, ef, W3[:, :128].T, W3[:, 128:].T, _pack(b3, g3, bb3),
      W4.T, _pack(b4, g4, bb4), W5.T, _pack(b5, g5, bb5),
      W6.T, _pack(b6, g6, bb6), W7[:, :128].T, W7[:, 128:].T,
      _pack(b7, g7, bb7), category_ids.reshape(_B, 1), cat_bias)

    return out.reshape(_B, _N, 512)
